# Initial kernel scaffold; baseline (speedup 1.0000x reference)
import jax
import jax.numpy as jnp
from jax.experimental import pallas as pl


def kernel(users, items, user_table, item_table, W1, b1, W2, b2, Wf, bf):
    raise NotImplementedError("write your pallas kernel here")


# speedup vs baseline: 2.7921x; 2.7921x over previous
import functools

import jax
import jax.numpy as jnp
from jax import lax
from jax.experimental import pallas as pl
from jax.experimental.pallas import tpu as pltpu
from jax.experimental.pallas import tpu_sc as plsc

EMB = 128
SC_CORES = 2
SC_SUBCORES = 16
SC_WORKERS = SC_CORES * SC_SUBCORES
GATHER_CHUNK = 256

MLP_TILE = 2048


def _sc_gather_pair(users, items, user_table, item_table):
    batch = users.shape[0]
    per_worker = batch // SC_WORKERS
    n_chunks = per_worker // GATHER_CHUNK
    mesh = plsc.VectorSubcoreMesh(core_axis_name="c", subcore_axis_name="s")

    @functools.partial(
        pl.kernel,
        mesh=mesh,
        out_type=(
            jax.ShapeDtypeStruct((batch, EMB), user_table.dtype),
            jax.ShapeDtypeStruct((batch, EMB), item_table.dtype),
        ),
        scratch_types=[
            pltpu.VMEM((GATHER_CHUNK,), jnp.int32),
            pltpu.VMEM((GATHER_CHUNK,), jnp.int32),
            pltpu.VMEM((GATHER_CHUNK, EMB), jnp.float32),
            pltpu.VMEM((GATHER_CHUNK, EMB), jnp.float32),
            pltpu.SemaphoreType.DMA,
            pltpu.SemaphoreType.DMA,
        ],
    )
    def gather_kernel(ut_hbm, it_hbm, u_idx_hbm, i_idx_hbm, ou_hbm, oi_hbm,
                      idx_u, idx_i, rows_u, rows_i, sem_u, sem_i):
        wid = lax.axis_index("s") * SC_CORES + lax.axis_index("c")
        base = wid * per_worker
        for p in range(n_chunks):
            off = base + p * GATHER_CHUNK
            pltpu.sync_copy(u_idx_hbm.at[pl.ds(off, GATHER_CHUNK)], idx_u)
            pltpu.sync_copy(i_idx_hbm.at[pl.ds(off, GATHER_CHUNK)], idx_i)
            cu = pltpu.async_copy(ut_hbm.at[idx_u], rows_u, sem_u)
            ci = pltpu.async_copy(it_hbm.at[idx_i], rows_i, sem_i)
            cu.wait()
            pltpu.sync_copy(rows_u, ou_hbm.at[pl.ds(off, GATHER_CHUNK)])
            ci.wait()
            pltpu.sync_copy(rows_i, oi_hbm.at[pl.ds(off, GATHER_CHUNK)])

    return gather_kernel(user_table, item_table, users, items)


def _mlp_body(u_ref, i_ref, w1u_ref, w1i_ref, b1_ref, w2_ref, b2_ref,
              wf_ref, bf_ref, o_ref):
    x = jnp.dot(u_ref[...], w1u_ref[...], preferred_element_type=jnp.float32)
    x = x + jnp.dot(i_ref[...], w1i_ref[...], preferred_element_type=jnp.float32)
    h1 = jnp.maximum(x + b1_ref[...], 0.0)
    h2 = jnp.maximum(
        jnp.dot(h1, w2_ref[...], preferred_element_type=jnp.float32)
        + b2_ref[...], 0.0)
    z = jnp.sum(h2 * wf_ref[...], axis=1, keepdims=True) + bf_ref[...]
    o_ref[...] = jax.nn.sigmoid(z)


def _tc_mlp(u_emb, i_emb, w1u_t, w1i_t, b1, w2_t, b2, wf, bf):
    batch = u_emb.shape[0]
    grid = (batch // MLP_TILE,)
    emb_spec = pl.BlockSpec((MLP_TILE, EMB), lambda i: (i, 0))
    full = lambda shape: pl.BlockSpec(shape, lambda i: (0, 0))
    return pl.pallas_call(
        _mlp_body,
        grid=grid,
        in_specs=[
            emb_spec,
            emb_spec,
            full((EMB, 256)),
            full((EMB, 256)),
            full((1, 256)),
            full((256, EMB)),
            full((1, EMB)),
            full((1, EMB)),
            full((1, 1)),
        ],
        out_specs=pl.BlockSpec((MLP_TILE, 1), lambda i: (i, 0)),
        out_shape=jax.ShapeDtypeStruct((batch, 1), jnp.float32),
        compiler_params=pltpu.CompilerParams(
            dimension_semantics=("parallel",)),
    )(u_emb, i_emb, w1u_t, w1i_t, b1, w2_t, b2, wf, bf)


def kernel(users, items, user_table, item_table, W1, b1, W2, b2, Wf, bf):
    u_emb, i_emb = _sc_gather_pair(users, items, user_table, item_table)
    w1u_t = W1[:, :EMB].T
    w1i_t = W1[:, EMB:].T
    return _tc_mlp(u_emb, i_emb, w1u_t, w1i_t,
                   b1.reshape(1, 256), W2.T, b2.reshape(1, EMB),
                   Wf, bf.reshape(1, 1))


# speedup vs baseline: 2.7945x; 1.0008x over previous
import functools

import jax
import jax.numpy as jnp
from jax import lax
from jax.experimental import pallas as pl
from jax.experimental.pallas import tpu as pltpu
from jax.experimental.pallas import tpu_sc as plsc

EMB = 128
SC_CORES = 2
SC_SUBCORES = 16
SC_WORKERS = SC_CORES * SC_SUBCORES
GATHER_CHUNK = 256

MLP_TILE = 2048


def _sc_gather_pair(users, items, user_table, item_table):
    batch = users.shape[0]
    per_worker = batch // SC_WORKERS
    n_chunks = per_worker // GATHER_CHUNK
    mesh = plsc.VectorSubcoreMesh(core_axis_name="c", subcore_axis_name="s")

    @functools.partial(
        pl.kernel,
        mesh=mesh,
        out_type=(
            jax.ShapeDtypeStruct((batch, EMB), user_table.dtype),
            jax.ShapeDtypeStruct((batch, EMB), item_table.dtype),
        ),
        scratch_types=[
            pltpu.VMEM((GATHER_CHUNK,), jnp.int32),
            pltpu.VMEM((GATHER_CHUNK,), jnp.int32),
            pltpu.VMEM((GATHER_CHUNK, EMB), jnp.float32),
            pltpu.VMEM((GATHER_CHUNK, EMB), jnp.float32),
            pltpu.SemaphoreType.DMA,
            pltpu.SemaphoreType.DMA,
        ],
    )
    def gather_kernel(ut_hbm, it_hbm, u_idx_hbm, i_idx_hbm, ou_hbm, oi_hbm,
                      idx_u, idx_i, rows_u, rows_i, sem_u, sem_i):
        wid = lax.axis_index("s") * SC_CORES + lax.axis_index("c")
        base = wid * per_worker
        for p in range(n_chunks):
            off = base + p * GATHER_CHUNK
            pltpu.sync_copy(u_idx_hbm.at[pl.ds(off, GATHER_CHUNK)], idx_u)
            pltpu.sync_copy(i_idx_hbm.at[pl.ds(off, GATHER_CHUNK)], idx_i)
            cu = pltpu.async_copy(ut_hbm.at[idx_u], rows_u, sem_u)
            ci = pltpu.async_copy(it_hbm.at[idx_i], rows_i, sem_i)
            cu.wait()
            pltpu.sync_copy(rows_u, ou_hbm.at[pl.ds(off, GATHER_CHUNK)])
            ci.wait()
            pltpu.sync_copy(rows_i, oi_hbm.at[pl.ds(off, GATHER_CHUNK)])

    return gather_kernel(user_table, item_table, users, items)


_CONTRACT_LAST = (((1,), (1,)), ((), ()))


def _mlp_body(u_ref, i_ref, w1u_ref, w1i_ref, b1_ref, w2_ref, b2_ref,
              wf_ref, bf_ref, o_ref):
    u = u_ref[...].astype(jnp.bfloat16)
    i = i_ref[...].astype(jnp.bfloat16)
    x = lax.dot_general(u, w1u_ref[...], _CONTRACT_LAST,
                        preferred_element_type=jnp.float32)
    x = x + lax.dot_general(i, w1i_ref[...], _CONTRACT_LAST,
                            preferred_element_type=jnp.float32)
    h1 = jnp.maximum(x + b1_ref[...], 0.0).astype(jnp.bfloat16)
    h2 = jnp.maximum(
        lax.dot_general(h1, w2_ref[...], _CONTRACT_LAST,
                        preferred_element_type=jnp.float32)
        + b2_ref[...], 0.0)
    z = jnp.sum(h2 * wf_ref[...], axis=1, keepdims=True) + bf_ref[...]
    o_ref[...] = jax.nn.sigmoid(z)


def _tc_mlp(u_emb, i_emb, w1u_t, w1i_t, b1, w2_t, b2, wf, bf):
    batch = u_emb.shape[0]
    grid = (batch // MLP_TILE,)
    emb_spec = pl.BlockSpec((MLP_TILE, EMB), lambda i: (i, 0))
    full = lambda shape: pl.BlockSpec(shape, lambda i: (0, 0))
    return pl.pallas_call(
        _mlp_body,
        grid=grid,
        in_specs=[
            emb_spec,
            emb_spec,
            full((256, EMB)),
            full((256, EMB)),
            full((1, 256)),
            full((EMB, 256)),
            full((1, EMB)),
            full((1, EMB)),
            full((1, 1)),
        ],
        out_specs=pl.BlockSpec((MLP_TILE, 1), lambda i: (i, 0)),
        out_shape=jax.ShapeDtypeStruct((batch, 1), jnp.float32),
        compiler_params=pltpu.CompilerParams(
            dimension_semantics=("parallel",)),
    )(u_emb, i_emb, w1u_t, w1i_t, b1, w2_t, b2, wf, bf)


def kernel(users, items, user_table, item_table, W1, b1, W2, b2, Wf, bf):
    u_emb, i_emb = _sc_gather_pair(users, items, user_table, item_table)
    w1u = W1[:, :EMB].astype(jnp.bfloat16)
    w1i = W1[:, EMB:].astype(jnp.bfloat16)
    return _tc_mlp(u_emb, i_emb, w1u, w1i,
                   b1.reshape(1, 256), W2.astype(jnp.bfloat16),
                   b2.reshape(1, EMB), Wf, bf.reshape(1, 1))


# speedup vs baseline: 3.1980x; 1.1444x over previous
import functools

import jax
import jax.numpy as jnp
from jax import lax
from jax.experimental import pallas as pl
from jax.experimental.pallas import tpu as pltpu
from jax.experimental.pallas import tpu_sc as plsc

EMB = 128
SC_CORES = 2
SC_SUBCORES = 16
SC_WORKERS = SC_CORES * SC_SUBCORES
GATHER_CHUNK = 256

MLP_TILE = 4096


def _sc_gather_pair(users, items, user_table, item_table):
    batch = users.shape[0]
    per_worker = batch // SC_WORKERS
    n_chunks = per_worker // GATHER_CHUNK
    mesh = plsc.VectorSubcoreMesh(core_axis_name="c", subcore_axis_name="s")

    @functools.partial(
        pl.kernel,
        mesh=mesh,
        out_type=(
            jax.ShapeDtypeStruct((batch, EMB), user_table.dtype),
            jax.ShapeDtypeStruct((batch, EMB), item_table.dtype),
        ),
        scratch_types=[
            pltpu.VMEM((GATHER_CHUNK,), jnp.int32),
            pltpu.VMEM((GATHER_CHUNK,), jnp.int32),
            pltpu.VMEM((GATHER_CHUNK, EMB), jnp.float32),
            pltpu.VMEM((GATHER_CHUNK, EMB), jnp.float32),
            pltpu.SemaphoreType.DMA,
            pltpu.SemaphoreType.DMA,
        ],
    )
    def gather_kernel(ut_hbm, it_hbm, u_idx_hbm, i_idx_hbm, ou_hbm, oi_hbm,
                      idx_u, idx_i, rows_u, rows_i, sem_u, sem_i):
        wid = lax.axis_index("s") * SC_CORES + lax.axis_index("c")
        base = wid * per_worker
        for p in range(n_chunks):
            off = base + p * GATHER_CHUNK
            pltpu.sync_copy(u_idx_hbm.at[pl.ds(off, GATHER_CHUNK)], idx_u)
            pltpu.sync_copy(i_idx_hbm.at[pl.ds(off, GATHER_CHUNK)], idx_i)
            cu = pltpu.async_copy(ut_hbm.at[idx_u], rows_u, sem_u)
            ci = pltpu.async_copy(it_hbm.at[idx_i], rows_i, sem_i)
            cu.wait()
            pltpu.sync_copy(rows_u, ou_hbm.at[pl.ds(off, GATHER_CHUNK)])
            ci.wait()
            pltpu.sync_copy(rows_i, oi_hbm.at[pl.ds(off, GATHER_CHUNK)])

    return gather_kernel(user_table, item_table, users, items)


_CONTRACT_LAST = (((1,), (1,)), ((), ()))


def _mlp_body(u_ref, i_ref, w1u_ref, w1i_ref, b1_ref, w2_ref, b2_ref,
              wf_ref, bf_ref, o_ref):
    u = u_ref[...].astype(jnp.bfloat16)
    i = i_ref[...].astype(jnp.bfloat16)
    x = lax.dot_general(u, w1u_ref[...], _CONTRACT_LAST,
                        preferred_element_type=jnp.float32)
    x = x + lax.dot_general(i, w1i_ref[...], _CONTRACT_LAST,
                            preferred_element_type=jnp.float32)
    h1 = jnp.maximum(x + b1_ref[...], 0.0).astype(jnp.bfloat16)
    h2 = jnp.maximum(
        lax.dot_general(h1, w2_ref[...], _CONTRACT_LAST,
                        preferred_element_type=jnp.float32)
        + b2_ref[...], 0.0)
    z = lax.dot_general(wf_ref[...], h2, _CONTRACT_LAST,
                        preferred_element_type=jnp.float32) + bf_ref[...]
    o_ref[...] = jax.nn.sigmoid(z)


def _tc_mlp(u_emb, i_emb, w1u_t, w1i_t, b1, w2_t, b2, wf, bf):
    batch = u_emb.shape[0]
    grid = (batch // MLP_TILE,)
    emb_spec = pl.BlockSpec((MLP_TILE, EMB), lambda i: (i, 0))
    full = lambda shape: pl.BlockSpec(shape, lambda i: (0, 0))
    return pl.pallas_call(
        _mlp_body,
        grid=grid,
        in_specs=[
            emb_spec,
            emb_spec,
            full((256, EMB)),
            full((256, EMB)),
            full((1, 256)),
            full((EMB, 256)),
            full((1, EMB)),
            full((1, EMB)),
            full((1, 1)),
        ],
        out_specs=pl.BlockSpec((1, MLP_TILE), lambda i: (0, i)),
        out_shape=jax.ShapeDtypeStruct((1, batch), jnp.float32),
        compiler_params=pltpu.CompilerParams(
            dimension_semantics=("parallel",)),
    )(u_emb, i_emb, w1u_t, w1i_t, b1, w2_t, b2, wf, bf)


def kernel(users, items, user_table, item_table, W1, b1, W2, b2, Wf, bf):
    u_emb, i_emb = _sc_gather_pair(users, items, user_table, item_table)
    w1u = W1[:, :EMB].astype(jnp.bfloat16)
    w1i = W1[:, EMB:].astype(jnp.bfloat16)
    out_row = _tc_mlp(u_emb, i_emb, w1u, w1i,
                      b1.reshape(1, 256), W2.astype(jnp.bfloat16),
                      b2.reshape(1, EMB), Wf, bf.reshape(1, 1))
    return out_row.reshape(users.shape[0], 1)
